# Initial kernel scaffold; baseline (speedup 1.0000x reference)
import jax
import jax.numpy as jnp
from jax.experimental import pallas as pl


def kernel(query, reference_points_cam, bev_mask):
    raise NotImplementedError("write your pallas kernel here")


# speedup vs baseline: 2.2750x; 2.2750x over previous
import functools

import jax
import jax.numpy as jnp
from jax import lax
from jax.experimental import pallas as pl
from jax.experimental.pallas import tpu as pltpu
from jax.experimental.pallas import tpu_sc as plsc

_NUM_CAMS = 6
_NUM_QUERY = 40000
_MAX_LEN = 10000
_EMBED = 256
_QPAD = 40960
_NBLK = _QPAD // 128
_IDXCAP = 10240
_ZVEC = _IDXCAP // 16
_DUMP = _IDXCAP - 16
_CHUNK = 80
_ROWS_PER_W = 2000
_NCHUNK = _ROWS_PER_W // _CHUNK
_SPAN = 2176
_CPC = _NUM_CAMS // 2
_RP_ZROW = _NUM_CAMS * _NUM_QUERY


def _tc_mask_body(bm_ref, addr_ref, cnorm_ref, lens_ref):
    bm = bm_ref[...]
    m = jnp.sum(bm, axis=1) > 0
    mf = m.astype(jnp.float32)
    mp = jnp.concatenate(
        [mf, jnp.zeros((_NUM_CAMS, _QPAD - _NUM_QUERY), jnp.float32)],
        axis=1)

    r = lax.broadcasted_iota(jnp.int32, (128, 128), 0)
    c = lax.broadcasted_iota(jnp.int32, (128, 128), 1)
    triu = (r <= c).astype(jnp.float32)
    s = jnp.dot(mp.reshape(_NUM_CAMS * _NBLK, 128), triu,
                preferred_element_type=jnp.float32)
    s = s.reshape(_NUM_CAMS, _NBLK, 128)

    tot = s[:, :, 127]
    rb = lax.broadcasted_iota(jnp.int32, (_NBLK, _NBLK), 0)
    cb = lax.broadcasted_iota(jnp.int32, (_NBLK, _NBLK), 1)
    sut = (rb < cb).astype(jnp.float32)
    boff = jnp.dot(tot, sut, preferred_element_type=jnp.float32)

    incl = (s + boff[:, :, None]).reshape(_NUM_CAMS, _QPAD)
    excl = incl - mp

    lane16 = jnp.tile(
        lax.broadcasted_iota(jnp.int32, (1, 16), 1),
        (1, _QPAD // 16)).astype(jnp.float32)
    hit = (mp > 0) & (excl < float(_MAX_LEN + 16))
    addr = jnp.where(hit, excl, float(_DUMP) + lane16)
    camoff = (lax.broadcasted_iota(jnp.int32, (_NUM_CAMS, 1), 0)
              % _CPC) * _IDXCAP
    addr_ref[...] = (addr.astype(jnp.int32) + camoff)[:, None, :]

    cnt = jnp.sum(mf, axis=0)
    cnorm_ref[...] = (1.0 / jnp.maximum(cnt, 1.0))[None, :]
    lens = jnp.minimum(incl[:, _QPAD - 1], float(_MAX_LEN)).astype(jnp.int32)
    lens_ref[...] = jnp.broadcast_to(lens[:, None, None],
                                     (_NUM_CAMS, 1, 128))


_tc_mask = pl.pallas_call(
    _tc_mask_body,
    out_shape=(
        jax.ShapeDtypeStruct((_NUM_CAMS, 1, _QPAD), jnp.int32),
        jax.ShapeDtypeStruct((1, _NUM_QUERY), jnp.float32),
        jax.ShapeDtypeStruct((_NUM_CAMS, 1, 128), jnp.int32),
    ),
)


def _sc_body(addr_hbm, vals_hbm, lens_hbm, q_hbm, rp_hbm,
             idx_out, q_out, rp_out,
             addr_v, vals_v, idx_buf, idxspan, qbuf, rpbuf, lenv,
             shared, sem_q, sem_r):
    core = lax.axis_index("c")
    sid = lax.axis_index("s")
    iota = lax.iota(jnp.int32, 16)

    @pl.when(sid < _CPC)
    def _compact():
        cl = sid
        cam = core * _CPC + cl
        pltpu.sync_copy(addr_hbm.at[cam, 0], addr_v)
        pltpu.sync_copy(vals_hbm, vals_v)
        pltpu.sync_copy(lens_hbm.at[cam, 0, pl.ds(0, 16)], lenv)
        n = lenv[...][0]

        pltpu.sync_copy(vals_v, shared.at[addr_v])
        pltpu.sync_copy(shared.at[pl.ds(cl * _IDXCAP, _IDXCAP)], idx_buf)

        def zbody(v, _):
            b = v * 16
            x = idx_buf[pl.ds(b, 16)]
            idx_buf[pl.ds(b, 16)] = jnp.where(b + iota < n, x, 0)
            return 0

        lax.fori_loop(0, _ZVEC, zbody, 0)

        pltpu.sync_copy(idx_buf.at[pl.ds(0, _MAX_LEN)], idx_out.at[cam, 0])
        pltpu.sync_copy(idx_buf, shared.at[pl.ds(cl * _IDXCAP, _IDXCAP)])

    plsc.subcore_barrier()

    @pl.when(sid < 15)
    def _gather():
        cl = sid // 5
        cam = core * _CPC + cl
        rstart = (sid % 5) * _ROWS_PER_W
        astart = (rstart // 128) * 128
        delta = rstart - astart
        pltpu.sync_copy(lens_hbm.at[cam, 0, pl.ds(0, 16)], lenv)
        n = lenv[...][0]
        pltpu.sync_copy(shared.at[pl.ds(cl * _IDXCAP + astart, _SPAN)],
                        idxspan)

        def gbody(c, _):
            off = rstart + c * _CHUNK
            loc = delta + c * _CHUNK
            cq = pltpu.async_copy(q_hbm.at[idxspan.at[pl.ds(loc, _CHUNK)]],
                                  qbuf, sem_q)
            rps = []
            for t in range(_CHUNK // 16):
                jvec = off + t * 16 + iota
                idxv = idxspan[pl.ds(loc + t * 16, 16)]
                riv = jnp.where(jvec < n, idxv + cam * _NUM_QUERY, _RP_ZROW)
                for l in range(16):
                    rps.append(pltpu.async_copy(
                        rp_hbm.at[pl.ds(riv[l] * 8, 8)],
                        rpbuf.at[t * 16 + l], sem_r))
            for cp in rps:
                cp.wait()
            cq.wait()

            valid = jnp.clip(n - off, 0, _CHUNK)

            def zb(r, _):
                for t in range(_EMBED // 16):
                    qbuf[r, pl.ds(t * 16, 16)] = jnp.zeros((16,),
                                                           jnp.float32)
                return 0

            lax.fori_loop(valid, _CHUNK, zb, 0)
            pltpu.sync_copy(qbuf, q_out.at[cam, pl.ds(off, _CHUNK)])
            pltpu.sync_copy(rpbuf, rp_out.at[cam, pl.ds(off, _CHUNK)])
            return 0

        lax.fori_loop(0, _NCHUNK, gbody, 0)


_sc_rebatch = functools.partial(
    pl.kernel,
    mesh=plsc.VectorSubcoreMesh(core_axis_name="c", subcore_axis_name="s"),
    out_type=(
        jax.ShapeDtypeStruct((_NUM_CAMS, 1, _MAX_LEN), jnp.int32),
        jax.ShapeDtypeStruct((_NUM_CAMS, _MAX_LEN, _EMBED), jnp.float32),
        jax.ShapeDtypeStruct((_NUM_CAMS, _MAX_LEN, 8), jnp.float32),
    ),
    scratch_types=[
        pltpu.VMEM((_QPAD,), jnp.int32),
        pltpu.VMEM((_QPAD,), jnp.int32),
        pltpu.VMEM((_IDXCAP,), jnp.int32),
        pltpu.VMEM((_SPAN,), jnp.int32),
        pltpu.VMEM((_CHUNK, _EMBED), jnp.float32),
        pltpu.VMEM((_CHUNK, 8), jnp.float32),
        pltpu.VMEM((16,), jnp.int32),
        pltpu.VMEM_SHARED((_CPC * _IDXCAP,), jnp.int32),
        pltpu.SemaphoreType.DMA,
        pltpu.SemaphoreType.DMA,
    ],
)(_sc_body)


@jax.jit
def kernel(query, reference_points_cam, bev_mask):
    bm = jnp.transpose(bev_mask.reshape(_NUM_CAMS, _NUM_QUERY, 4),
                       (0, 2, 1))
    addr, cnorm, lens3d = _tc_mask(bm)
    q2d = query.reshape(_NUM_QUERY, _EMBED)
    rp_flat = jnp.concatenate(
        [reference_points_cam.reshape(_NUM_CAMS * _NUM_QUERY, 8),
         jnp.zeros((8, 8), jnp.float32)], axis=0).reshape(-1)
    vals = jnp.arange(_QPAD, dtype=jnp.int32)
    indexes, qrb, rprb = _sc_rebatch(addr, vals, lens3d, q2d, rp_flat)
    return (
        indexes.reshape(_NUM_CAMS, _MAX_LEN),
        lens3d[:, 0, 0],
        qrb,
        rprb.reshape(_NUM_CAMS, _MAX_LEN, 4, 2),
        cnorm.reshape(1, _NUM_QUERY, 1),
    )


# speedup vs baseline: 2.2963x; 1.0093x over previous
import functools

import jax
import jax.numpy as jnp
from jax import lax
from jax.experimental import pallas as pl
from jax.experimental.pallas import tpu as pltpu
from jax.experimental.pallas import tpu_sc as plsc

_NUM_CAMS = 6
_NUM_QUERY = 40000
_MAX_LEN = 10000
_EMBED = 256
_QPAD = 40960
_NBLK = _QPAD // 128
_IDXCAP = 10240
_ZVEC = _IDXCAP // 16
_DUMP = _IDXCAP - 16
_CHUNK = 80
_ROWS_PER_W = 2000
_NCHUNK = _ROWS_PER_W // _CHUNK
_SPAN = 2176
_CPC = _NUM_CAMS // 2
_ASLICE = _QPAD // 5
_RP_ZROW = _NUM_CAMS * _NUM_QUERY


def _tc_mask_body(bm_ref, addr_ref, cnorm_ref, lens_ref):
    bm = bm_ref[...]
    m = jnp.sum(bm, axis=1) > 0
    mf = m.astype(jnp.float32)
    mp = jnp.concatenate(
        [mf, jnp.zeros((_NUM_CAMS, _QPAD - _NUM_QUERY), jnp.float32)],
        axis=1)

    r = lax.broadcasted_iota(jnp.int32, (128, 128), 0)
    c = lax.broadcasted_iota(jnp.int32, (128, 128), 1)
    triu = (r <= c).astype(jnp.float32)
    s = jnp.dot(mp.reshape(_NUM_CAMS * _NBLK, 128), triu,
                preferred_element_type=jnp.float32)
    s = s.reshape(_NUM_CAMS, _NBLK, 128)

    tot = s[:, :, 127]
    rb = lax.broadcasted_iota(jnp.int32, (_NBLK, _NBLK), 0)
    cb = lax.broadcasted_iota(jnp.int32, (_NBLK, _NBLK), 1)
    sut = (rb < cb).astype(jnp.float32)
    boff = jnp.dot(tot, sut, preferred_element_type=jnp.float32)

    incl = (s + boff[:, :, None]).reshape(_NUM_CAMS, _QPAD)
    excl = incl - mp

    lane16 = jnp.tile(
        lax.broadcasted_iota(jnp.int32, (1, 16), 1),
        (1, _QPAD // 16)).astype(jnp.float32)
    hit = (mp > 0) & (excl < float(_MAX_LEN + 16))
    addr = jnp.where(hit, excl, float(_DUMP) + lane16)
    camoff = (lax.broadcasted_iota(jnp.int32, (_NUM_CAMS, 1), 0)
              % _CPC) * _IDXCAP
    addr_ref[...] = (addr.astype(jnp.int32) + camoff)[:, None, :]

    cnt = jnp.sum(mf, axis=0)
    cnorm_ref[...] = (1.0 / jnp.maximum(cnt, 1.0))[None, :]
    lens = jnp.minimum(incl[:, _QPAD - 1], float(_MAX_LEN)).astype(jnp.int32)
    lens_ref[...] = jnp.broadcast_to(lens[:, None, None],
                                     (_NUM_CAMS, 1, 128))


_tc_mask = pl.pallas_call(
    _tc_mask_body,
    out_shape=(
        jax.ShapeDtypeStruct((_NUM_CAMS, 1, _QPAD), jnp.int32),
        jax.ShapeDtypeStruct((1, _NUM_QUERY), jnp.float32),
        jax.ShapeDtypeStruct((_NUM_CAMS, 1, 128), jnp.int32),
    ),
)


def _sc_body(addr_hbm, vals_hbm, lens_hbm, q_hbm, rp_hbm,
             idx_out, q_out, rp_out,
             addr_v, vals_v, idx_buf, idxspan, qbuf0, qbuf1, rpbuf0, rpbuf1,
             lenv, shared, sem_q, sem_q2, sem_r, sem_r2):
    core = lax.axis_index("c")
    sid = lax.axis_index("s")
    iota = lax.iota(jnp.int32, 16)

    @pl.when(sid < 15)
    def _scatter():
        cl = sid // 5
        cam = core * _CPC + cl
        k = sid % 5
        pltpu.sync_copy(addr_hbm.at[cam, 0, pl.ds(k * _ASLICE, _ASLICE)],
                        addr_v)
        pltpu.sync_copy(vals_hbm.at[pl.ds(k * _ASLICE, _ASLICE)], vals_v)
        pltpu.sync_copy(vals_v, shared.at[addr_v])

    plsc.subcore_barrier()

    @pl.when(sid < _CPC)
    def _finish():
        cl = sid
        cam = core * _CPC + cl
        pltpu.sync_copy(lens_hbm.at[cam, 0, pl.ds(0, 16)], lenv)
        n = lenv[...][0]
        pltpu.sync_copy(shared.at[pl.ds(cl * _IDXCAP, _IDXCAP)], idx_buf)

        def zbody(v, _):
            b = v * 16
            x = idx_buf[pl.ds(b, 16)]
            idx_buf[pl.ds(b, 16)] = jnp.where(b + iota < n, x, 0)
            return 0

        lax.fori_loop(0, _ZVEC, zbody, 0)

        pltpu.sync_copy(idx_buf.at[pl.ds(0, _MAX_LEN)], idx_out.at[cam, 0])
        pltpu.sync_copy(idx_buf, shared.at[pl.ds(cl * _IDXCAP, _IDXCAP)])

    plsc.subcore_barrier()

    @pl.when(sid < 15)
    def _gather():
        cl = sid // 5
        cam = core * _CPC + cl
        rstart = (sid % 5) * _ROWS_PER_W
        astart = (rstart // 128) * 128
        delta = rstart - astart
        pltpu.sync_copy(lens_hbm.at[cam, 0, pl.ds(0, 16)], lenv)
        n = lenv[...][0]
        pltpu.sync_copy(shared.at[pl.ds(cl * _IDXCAP + astart, _SPAN)],
                        idxspan)

        def issue_q(c, qb, sq):
            loc = delta + c * _CHUNK
            return pltpu.async_copy(
                q_hbm.at[idxspan.at[pl.ds(loc, _CHUNK)]], qb, sq)

        def do_rp(c, rpb, sr):
            off = rstart + c * _CHUNK
            loc = delta + c * _CHUNK
            rps = []
            for t in range(_CHUNK // 16):
                jvec = off + t * 16 + iota
                idxv = idxspan[pl.ds(loc + t * 16, 16)]
                riv = jnp.where(jvec < n, idxv + cam * _NUM_QUERY, _RP_ZROW)
                for l in range(16):
                    rps.append(pltpu.async_copy(
                        rp_hbm.at[pl.ds(riv[l] * 8, 8)],
                        rpb.at[t * 16 + l], sr))
            for cp in rps:
                cp.wait()

        def finish(c, cpq, qb, rpb):
            cpq.wait()
            off = rstart + c * _CHUNK
            valid = jnp.clip(n - off, 0, _CHUNK)

            def zb(r, _):
                for t in range(_EMBED // 16):
                    qb[r, pl.ds(t * 16, 16)] = jnp.zeros((16,), jnp.float32)
                return 0

            lax.fori_loop(valid, _CHUNK, zb, 0)
            pltpu.sync_copy(qb, q_out.at[cam, pl.ds(off, _CHUNK)])
            pltpu.sync_copy(rpb, rp_out.at[cam, pl.ds(off, _CHUNK)])

        def body2(g, _):
            c0 = 2 * g
            c1 = 2 * g + 1
            cp0 = issue_q(c0, qbuf0, sem_q)
            cp1 = issue_q(c1, qbuf1, sem_q2)
            do_rp(c0, rpbuf0, sem_r)
            finish(c0, cp0, qbuf0, rpbuf0)
            do_rp(c1, rpbuf1, sem_r2)
            finish(c1, cp1, qbuf1, rpbuf1)
            return 0

        lax.fori_loop(0, _NCHUNK // 2, body2, 0)
        ct = _NCHUNK - 1
        cpt = issue_q(ct, qbuf0, sem_q)
        do_rp(ct, rpbuf0, sem_r)
        finish(ct, cpt, qbuf0, rpbuf0)


_sc_rebatch = functools.partial(
    pl.kernel,
    mesh=plsc.VectorSubcoreMesh(core_axis_name="c", subcore_axis_name="s"),
    out_type=(
        jax.ShapeDtypeStruct((_NUM_CAMS, 1, _MAX_LEN), jnp.int32),
        jax.ShapeDtypeStruct((_NUM_CAMS, _MAX_LEN, _EMBED), jnp.float32),
        jax.ShapeDtypeStruct((_NUM_CAMS, _MAX_LEN, 8), jnp.float32),
    ),
    scratch_types=[
        pltpu.VMEM((_ASLICE,), jnp.int32),
        pltpu.VMEM((_ASLICE,), jnp.int32),
        pltpu.VMEM((_IDXCAP,), jnp.int32),
        pltpu.VMEM((_SPAN,), jnp.int32),
        pltpu.VMEM((_CHUNK, _EMBED), jnp.float32),
        pltpu.VMEM((_CHUNK, _EMBED), jnp.float32),
        pltpu.VMEM((_CHUNK, 8), jnp.float32),
        pltpu.VMEM((_CHUNK, 8), jnp.float32),
        pltpu.VMEM((16,), jnp.int32),
        pltpu.VMEM_SHARED((_CPC * _IDXCAP,), jnp.int32),
        pltpu.SemaphoreType.DMA,
        pltpu.SemaphoreType.DMA,
        pltpu.SemaphoreType.DMA,
        pltpu.SemaphoreType.DMA,
    ],
)(_sc_body)


@jax.jit
def kernel(query, reference_points_cam, bev_mask):
    bm = jnp.transpose(bev_mask.reshape(_NUM_CAMS, _NUM_QUERY, 4),
                       (0, 2, 1))
    addr, cnorm, lens3d = _tc_mask(bm)
    q2d = query.reshape(_NUM_QUERY, _EMBED)
    rp_flat = jnp.concatenate(
        [reference_points_cam.reshape(_NUM_CAMS * _NUM_QUERY, 8),
         jnp.zeros((8, 8), jnp.float32)], axis=0).reshape(-1)
    vals = jnp.arange(_QPAD, dtype=jnp.int32)
    indexes, qrb, rprb = _sc_rebatch(addr, vals, lens3d, q2d, rp_flat)
    return (
        indexes.reshape(_NUM_CAMS, _MAX_LEN),
        lens3d[:, 0, 0],
        qrb,
        rprb.reshape(_NUM_CAMS, _MAX_LEN, 4, 2),
        cnorm.reshape(1, _NUM_QUERY, 1),
    )
